# Initial kernel scaffold; baseline (speedup 1.0000x reference)
import jax
import jax.numpy as jnp
from jax.experimental import pallas as pl


def kernel(oimage, aimage, olabel, alabel, oconf, aconf, prediction, cur_step):
    raise NotImplementedError("write your pallas kernel here")


# speedup vs baseline: 6.6517x; 6.6517x over previous
import jax
import jax.numpy as jnp
from jax import lax
from jax.experimental import pallas as pl
from jax.experimental.pallas import tpu as pltpu
from jax.experimental.pallas import tpu_sc as plsc

_B, _C, _H = 8, 3, 512
_PH = 32
_G = 16
_L = 256
_TOPK = 16
_NC = 4
_AGE = 1.0

_QR = 128
_NIR = _B * _C * _H
_NLR = _B * _H


def _stats_kernel(pred_ref, olab_ref, oconf_ref, aconf_ref,
                  tk_ref, dv_ref, sv_ref):
    pred = pred_ref[0]
    olab = olab_ref[0]
    oconf = oconf_ref[0]
    aconf = aconf_ref[0]

    m = jnp.max(pred, axis=0)
    e = jnp.exp(pred - m[None, :, :])
    sinv = 1.0 / jnp.sum(e, axis=0)
    dice_acc = jnp.float32(0.0)
    for c in range(_NC):
        p_c = e[c] * sinv
        t_c = (olab == c).astype(jnp.float32)
        inter = jnp.sum(p_c * t_c)
        union = jnp.sum(p_c) + jnp.sum(t_c)
        dice_acc = dice_acc + 2.0 * inter / (union + 1e-5)
    dice = dice_acc / _NC
    loss = 1.0 - dice
    smask = loss < _AGE
    w = 1.0 - loss / (_AGE + 1e-5)
    tk = jnp.minimum(jnp.float32(_TOPK),
                     jnp.abs(jnp.trunc(_TOPK * w))).astype(jnp.int32)

    l_i = lax.broadcasted_iota(jnp.int32, (_L, _H), 0)
    w_i = lax.broadcasted_iota(jnp.int32, (_L, _H), 1)
    A = (w_i // _PH == l_i // _G).astype(jnp.float32)
    Bm = (w_i // _PH == l_i % _G).astype(jnp.float32)

    def patch_sums(x):
        rows = jnp.dot(A, x, preferred_element_type=jnp.float32,
                       precision=lax.Precision.HIGHEST)
        return jnp.sum(rows * Bm, axis=1, keepdims=True)

    osum = patch_sums(oconf)
    asum = patch_sums(aconf)

    pi = lax.broadcasted_iota(jnp.int32, (_L, _L), 0)
    qi = lax.broadcasted_iota(jnp.int32, (_L, _L), 1)
    eye = (pi == qi).astype(jnp.float32)

    def ranks_col(key_col, descending):
        kp = key_col
        kq = lax.dot_general(key_col, eye, (((0,), (0,)), ((), ())),
                             preferred_element_type=jnp.float32,
                             precision=lax.Precision.HIGHEST)
        before = (descending & (kq > kp)) | (
            jnp.logical_not(descending) & (kq < kp))
        tie = (kq == kp) & (qi < pi)
        return jnp.sum((before | tie).astype(jnp.int32), axis=1,
                       keepdims=True)

    rank_o = ranks_col(osum, smask)
    rank_a = ranks_col(asum, jnp.logical_not(smask))

    j16 = lax.broadcasted_iota(jnp.int32, (_L, _G), 1)
    p_col = lax.broadcasted_iota(jnp.int32, (_L, _G), 0)
    d_row = jnp.sum(jnp.where(rank_o == j16, p_col, 0), axis=0,
                    keepdims=True)
    s_row = jnp.sum(jnp.where(rank_a == j16, p_col, 0), axis=0,
                    keepdims=True)

    tk_ref[0] = jnp.full((1, 16), tk, jnp.int32)
    dv_ref[0] = d_row
    sv_ref[0] = s_row


def _stage1(olabel, oconf, aconf, prediction):
    return pl.pallas_call(
        _stats_kernel,
        grid=(_B,),
        in_specs=[
            pl.BlockSpec((1, _NC, _H, _H), lambda b: (b, 0, 0, 0)),
            pl.BlockSpec((1, _H, _H), lambda b: (b, 0, 0)),
            pl.BlockSpec((1, _H, _H), lambda b: (b, 0, 0)),
            pl.BlockSpec((1, _H, _H), lambda b: (b, 0, 0)),
        ],
        out_specs=[
            pl.BlockSpec((1, 1, 16), lambda b: (b, 0, 0)),
            pl.BlockSpec((1, 1, 16), lambda b: (b, 0, 0)),
            pl.BlockSpec((1, 1, 16), lambda b: (b, 0, 0)),
        ],
        out_shape=[
            jax.ShapeDtypeStruct((_B, 1, 16), jnp.int32),
            jax.ShapeDtypeStruct((_B, 1, 16), jnp.int32),
            jax.ShapeDtypeStruct((_B, 1, 16), jnp.int32),
        ],
    )(prediction, olabel, oconf, aconf)


def _merge_patch(pbuf, abuf, pr_local, pc, sc):
    def body(r, _):
        src0 = sc * _PH
        dst0 = pc * _PH
        row_d = pr_local * _PH + r
        pbuf[row_d, pl.ds(dst0, 16)] = abuf[r, pl.ds(src0, 16)]
        pbuf[row_d, pl.ds(dst0 + 16, 16)] = abuf[r, pl.ds(src0 + 16, 16)]
        return 0
    lax.fori_loop(0, _PH, body, 0)


def _sc_assemble_body(oi, ai, ol, al, oc, ac, tkv, dv, sv,
                      outi, outl, outc,
                      pbuf, abuf, tk_v, d_v, s_v):
    c = lax.axis_index("c")
    s = lax.axis_index("s")
    tile = c * 16 + s
    b = tile // 4

    pltpu.sync_copy(tkv.at[pl.ds(b * 16, 16)], tk_v)
    pltpu.sync_copy(dv.at[pl.ds(b * 16, 16)], d_v)
    pltpu.sync_copy(sv.at[pl.ds(b * 16, 16)], s_v)
    tkb = tk_v[...][0]
    dvec = d_v[...]
    svec = s_v[...]

    def do_unit(osrc, asrc, dst, row0, arow_base):
        pltpu.sync_copy(osrc.at[pl.ds(row0, _QR)], pbuf)
        q = (row0 % _H) // _QR
        for j in range(_TOPK):
            d_j = dvec[j]
            s_j = svec[j]
            pr = d_j // _G

            @pl.when((j < tkb) & (pr // 4 == q))
            def _():
                sr = s_j // _G
                pltpu.sync_copy(asrc.at[pl.ds(arow_base + sr * _PH, _PH)],
                                abuf)
                _merge_patch(pbuf, abuf, pr % 4, d_j % _G, s_j % _G)

        pltpu.sync_copy(pbuf, dst.at[pl.ds(row0, _QR)])

    for m in range(3):
        u = tile * 3 + m
        ch = (u % 12) // 4
        q = u % 4
        row0 = (b * _C + ch) * _H + q * _QR
        do_unit(oi, ai, outi, row0, (b * _C + ch) * _H)

    qlc = tile % 4
    lrow0 = b * _H + qlc * _QR
    do_unit(ol, al, outl, lrow0, b * _H)
    do_unit(oc, ac, outc, lrow0, b * _H)


def kernel(oimage, aimage, olabel, alabel, oconf, aconf, prediction, cur_step):
    olabel = olabel.astype(jnp.int32)
    alabel = alabel.astype(jnp.int32)

    tk3, dv3, sv3 = _stage1(olabel, oconf, aconf, prediction)
    tkv = tk3.reshape(_B * 16)
    dv = dv3.reshape(_B * 16)
    sv = sv3.reshape(_B * 16)

    oi_t = oimage.reshape(_NIR, _H)
    ai_t = aimage.reshape(_NIR, _H)
    ol_t = lax.bitcast_convert_type(olabel, jnp.float32).reshape(_NLR, _H)
    al_t = lax.bitcast_convert_type(alabel, jnp.float32).reshape(_NLR, _H)
    oc_t = oconf.reshape(_NLR, _H)
    ac_t = aconf.reshape(_NLR, _H)

    sc_call = pl.kernel(
        _sc_assemble_body,
        out_type=[
            jax.ShapeDtypeStruct((_NIR, _H), jnp.float32),
            jax.ShapeDtypeStruct((_NLR, _H), jnp.float32),
            jax.ShapeDtypeStruct((_NLR, _H), jnp.float32),
        ],
        mesh=plsc.VectorSubcoreMesh(core_axis_name="c", subcore_axis_name="s"),
        scratch_types=[
            pltpu.VMEM((_QR, _H), jnp.float32),
            pltpu.VMEM((_PH, _H), jnp.float32),
            pltpu.VMEM((16,), jnp.int32),
            pltpu.VMEM((16,), jnp.int32),
            pltpu.VMEM((16,), jnp.int32),
        ],
    )

    outi, outl, outc = sc_call(oi_t, ai_t, ol_t, al_t, oc_t, ac_t,
                               tkv, dv, sv)

    out_img = outi.reshape(_B, _C, _H, _H)
    out_lab = lax.bitcast_convert_type(outl.reshape(_B, _H, _H), jnp.int32)
    out_conf = outc.reshape(_B, _H, _H)
    return out_img, out_lab, out_conf
